# baseline (speedup 1.0000x reference)
import functools
import jax
import jax.numpy as jnp
from jax import lax
from jax.experimental import pallas as pl
from jax.experimental.pallas import tpu as pltpu
from jax.experimental.pallas import tpu_sc as plsc

N = 10000
E = 320000
D = 128
HEADS = 8
NUM_GRAPHS = 16

NC = 2
NS = 16
NW = NC * NS
EPW = E // NW
B = 100
NBLK = EPW // B
BG = 50
NBLKG = EPW // BG

f32 = jnp.float32
i32 = jnp.int32

_mesh = plsc.VectorSubcoreMesh(core_axis_name="c", subcore_axis_name="s")
_sc_params = pltpu.CompilerParams(use_tc_tiling_on_sc=False)


def _wid():
    return lax.axis_index("s") * NC + lax.axis_index("c")


def _gwait(src, dst, sem):
    pltpu.make_async_copy(src, dst, sem).wait()


@functools.partial(
    pl.kernel,
    out_type=jax.ShapeDtypeStruct((NC, N, 16), f32),
    mesh=_mesh,
    compiler_params=_sc_params,
    scratch_types=[
        pltpu.VMEM_SHARED((N, 16), f32),
        pltpu.VMEM((NBLK, B), i32),
        pltpu.VMEM((B, 16), f32),
        pltpu.SemaphoreType.DMA,
    ],
)
def _sc_deg(dst_hbm, z_hbm, out_hbm, acc, didx, onesb, sem):
    c = lax.axis_index("c")
    s = lax.axis_index("s")
    wid = _wid()

    @pl.when(s == 0)
    def _():
        pltpu.sync_copy(z_hbm, acc)

    pltpu.sync_copy(dst_hbm.at[wid], didx)
    row = jnp.where(lax.iota(i32, 16) == 0, 1.0, 0.0).astype(f32)

    def bstep(j, _):
        onesb[j] = row
        return 0

    lax.fori_loop(0, B, bstep, 0)
    plsc.subcore_barrier()

    def step(i, _):
        for t in range(10):
            pltpu.async_copy(onesb, acc.at[didx.at[i * 10 + t]], sem,
                             add=True)
        for t in range(10):
            _gwait(onesb, acc.at[didx.at[i * 10 + t]], sem)
        return 0

    lax.fori_loop(0, NBLK // 10, step, 0)
    plsc.subcore_barrier()

    @pl.when(s == 0)
    def _():
        pltpu.sync_copy(acc, out_hbm.at[c])


@functools.partial(
    pl.kernel,
    out_type=jax.ShapeDtypeStruct((NC, N, D), f32),
    mesh=_mesh,
    compiler_params=_sc_params,
    scratch_types=[
        pltpu.VMEM_SHARED((N, D), f32),
        pltpu.VMEM((NBLK, B), i32),
        pltpu.VMEM((NBLK, B), i32),
        pltpu.VMEM((B, D), f32),
        pltpu.VMEM((B, D), f32),
        pltpu.SemaphoreType.DMA,
        pltpu.SemaphoreType.DMA,
    ],
)
def _sc_rows(tbl_hbm, src_hbm, dst_hbm, z_hbm, out_hbm, acc, sidx, didx,
             rowsA, rowsB, semA, semB):
    c = lax.axis_index("c")
    s = lax.axis_index("s")
    wid = _wid()

    @pl.when(s == 0)
    def _():
        pltpu.sync_copy(z_hbm, acc)

    pltpu.sync_copy(src_hbm.at[wid], sidx)
    pltpu.sync_copy(dst_hbm.at[wid], didx)
    plsc.subcore_barrier()

    pltpu.async_copy(tbl_hbm.at[sidx.at[0]], rowsA, semA)

    def step(p, _):
        b0 = 2 * p
        b1 = b0 + 1
        pltpu.async_copy(tbl_hbm.at[sidx.at[b1]], rowsB, semB)
        _gwait(tbl_hbm.at[sidx.at[b0]], rowsA, semA)
        pltpu.sync_copy(rowsA, acc.at[didx.at[b0]], add=True)

        @pl.when(b0 + 2 < NBLK)
        def _():
            pltpu.async_copy(tbl_hbm.at[sidx.at[b0 + 2]], rowsA, semA)

        _gwait(tbl_hbm.at[sidx.at[b1]], rowsB, semB)
        pltpu.sync_copy(rowsB, acc.at[didx.at[b1]], add=True)
        return 0

    lax.fori_loop(0, NBLK // 2, step, 0)
    plsc.subcore_barrier()

    @pl.when(s == 0)
    def _():
        pltpu.sync_copy(acc, out_hbm.at[c])


@functools.partial(
    pl.kernel,
    out_type=(
        jax.ShapeDtypeStruct((NC, N, 16), f32),
        jax.ShapeDtypeStruct((E, 16), f32),
    ),
    mesh=_mesh,
    compiler_params=_sc_params,
    scratch_types=[
        pltpu.VMEM_SHARED((N, 16), f32),
        pltpu.VMEM((NBLK, B), i32),
        pltpu.VMEM((NBLK, B), i32),
        pltpu.VMEM((B, 16), f32),
        pltpu.VMEM((B, 16), f32),
        pltpu.VMEM((B, 16), f32),
        pltpu.VMEM((B, 16), f32),
        pltpu.VMEM((B, 16), f32),
        pltpu.SemaphoreType.DMA,
        pltpu.SemaphoreType.DMA,
        pltpu.SemaphoreType.DMA,
        pltpu.SemaphoreType.DMA,
    ],
)
def _sc_gata(alcs_hbm, alcd_hbm, src_hbm, dst_hbm, z_hbm, den_hbm, ex_hbm,
             den, sidx, didx, rsA, rdA, rsB, rdB, exb, semSA, semDA, semSB,
             semDB):
    c = lax.axis_index("c")
    s = lax.axis_index("s")
    wid = _wid()

    @pl.when(s == 0)
    def _():
        pltpu.sync_copy(z_hbm, den)

    pltpu.sync_copy(src_hbm.at[wid], sidx)
    pltpu.sync_copy(dst_hbm.at[wid], didx)
    plsc.subcore_barrier()

    lo_mask = lax.iota(i32, 16) < 8

    def start(b, rs, rd, semS, semD):
        pltpu.async_copy(alcs_hbm.at[sidx.at[b]], rs, semS)
        pltpu.async_copy(alcd_hbm.at[didx.at[b]], rd, semD)

    def finish(b, rs, rd, semS, semD):
        _gwait(alcs_hbm.at[sidx.at[b]], rs, semS)
        _gwait(alcd_hbm.at[didx.at[b]], rd, semD)

        def estep(j, _):
            z = rs[j] + rd[j]
            z = jnp.maximum(z, 0.2 * z)
            exv = jnp.exp(z)
            exb[j] = jnp.where(lo_mask, exv, 0.0)
            return 0

        lax.fori_loop(0, B, estep, 0)
        pltpu.sync_copy(exb, ex_hbm.at[pl.ds((wid * NBLK + b) * B, B)])
        pltpu.sync_copy(exb, den.at[didx.at[b]], add=True)

    start(0, rsA, rdA, semSA, semDA)

    def step(p, _):
        b0 = 2 * p
        b1 = b0 + 1
        start(b1, rsB, rdB, semSB, semDB)
        finish(b0, rsA, rdA, semSA, semDA)

        @pl.when(b0 + 2 < NBLK)
        def _():
            start(b0 + 2, rsA, rdA, semSA, semDA)

        finish(b1, rsB, rdB, semSB, semDB)
        return 0

    lax.fori_loop(0, NBLK // 2, step, 0)
    plsc.subcore_barrier()

    @pl.when(s == 0)
    def _():
        pltpu.sync_copy(den, den_hbm.at[c])


@functools.partial(
    pl.kernel,
    out_type=jax.ShapeDtypeStruct((E, D), f32),
    mesh=_mesh,
    compiler_params=_sc_params,
    scratch_types=[
        pltpu.VMEM((NBLKG, BG), i32),
        pltpu.VMEM((NBLKG, BG), i32),
        pltpu.VMEM((BG, HEADS * D // 2), f32),
        pltpu.VMEM((BG, HEADS * D // 2), f32),
        pltpu.VMEM((BG, 16), f32),
        pltpu.VMEM((BG, 16), f32),
        pltpu.VMEM((BG, 16), f32),
        pltpu.VMEM((BG, 16), f32),
        pltpu.VMEM((BG, D), f32),
        pltpu.SemaphoreType.DMA,
        pltpu.SemaphoreType.DMA,
        pltpu.SemaphoreType.DMA,
        pltpu.SemaphoreType.DMA,
        pltpu.SemaphoreType.DMA,
        pltpu.SemaphoreType.DMA,
    ],
)
def _sc_gatb(xh_hbm, ex_hbm, rdenc_hbm, src_hbm, dst_hbm, wr_hbm,
             sidx, didx, xrA, xrB, exA, exB, rdA, rdB, outb,
             semXA, semXB, semEA, semEB, semRA, semRB):
    wid = _wid()

    pltpu.sync_copy(src_hbm.at[wid], sidx)
    pltpu.sync_copy(dst_hbm.at[wid], didx)

    def start(b, xr, exb, rdb, semX, semE, semR):
        pltpu.async_copy(xh_hbm.at[sidx.at[b]], xr, semX)
        pltpu.async_copy(ex_hbm.at[pl.ds((wid * NBLKG + b) * BG, BG)], exb,
                         semE)
        pltpu.async_copy(rdenc_hbm.at[didx.at[b]], rdb, semR)

    def finish(b, xr, exb, rdb, semX, semE, semR):
        _gwait(xh_hbm.at[sidx.at[b]], xr, semX)
        _gwait(ex_hbm.at[pl.ds((wid * NBLKG + b) * BG, BG)], exb, semE)
        _gwait(rdenc_hbm.at[didx.at[b]], rdb, semR)

        def estep(j, _):
            cvec = exb[j] * rdb[j]
            accs = [jnp.zeros((16,), f32) for _ in range(D // 16)]
            for h in range(HEADS):
                cv = jnp.full((16,), cvec[h], f32)
                for k2 in range(D // 32):
                    w = xr[j, pl.ds(h * (D // 2) + k2 * 16, 16)]
                    wi = lax.bitcast_convert_type(w, i32)
                    va = lax.bitcast_convert_type(wi << 16, f32)
                    vb = lax.bitcast_convert_type(wi & jnp.int32(-65536), f32)
                    accs[2 * k2] = accs[2 * k2] + cv * va
                    accs[2 * k2 + 1] = accs[2 * k2 + 1] + cv * vb
            for k in range(D // 16):
                outb[j, pl.ds(k * 16, 16)] = accs[k]
            return 0

        lax.fori_loop(0, BG, estep, 0)
        pltpu.sync_copy(outb, wr_hbm.at[pl.ds((wid * NBLKG + b) * BG, BG)])

    start(0, xrA, exA, rdA, semXA, semEA, semRA)

    def step(p, _):
        b0 = 2 * p
        b1 = b0 + 1
        start(b1, xrB, exB, rdB, semXB, semEB, semRB)
        finish(b0, xrA, exA, rdA, semXA, semEA, semRA)

        @pl.when(b0 + 2 < NBLKG)
        def _():
            start(b0 + 2, xrA, exA, rdA, semXA, semEA, semRA)

        finish(b1, xrB, exB, rdB, semXB, semEB, semRB)
        return 0

    lax.fori_loop(0, NBLKG // 2, step, 0)


NB = 2000
NBT = N // NB


def _t1_body(d0_ref, d1_ref, x_ref, w_ref, hp_ref, dinv_ref, rcnt_ref):
    cnt = d0_ref[...][:, :1] + d1_ref[...][:, :1]
    dinv = lax.rsqrt(cnt + 1.0)
    rcnt_ref[...] = 1.0 / jnp.maximum(cnt, 1.0)
    h = jnp.dot(x_ref[...], w_ref[...], preferred_element_type=f32)
    hp_ref[...] = h * dinv
    dinv_ref[...] = dinv


def _t1(d0, d1, x, w_gcn):
    return pl.pallas_call(
        _t1_body,
        grid=(NBT,),
        in_specs=[
            pl.BlockSpec((NB, 16), lambda i: (i, 0)),
            pl.BlockSpec((NB, 16), lambda i: (i, 0)),
            pl.BlockSpec((NB, D), lambda i: (i, 0)),
            pl.BlockSpec((D, D), lambda i: (0, 0)),
        ],
        out_specs=[
            pl.BlockSpec((NB, D), lambda i: (i, 0)),
            pl.BlockSpec((NB, 1), lambda i: (i, 0)),
            pl.BlockSpec((NB, 1), lambda i: (i, 0)),
        ],
        out_shape=[
            jax.ShapeDtypeStruct((N, D), f32),
            jax.ShapeDtypeStruct((N, 1), f32),
            jax.ShapeDtypeStruct((N, 1), f32),
        ],
    )(d0, d1, x, w_gcn)


def _rne_bf16(a):
    u = lax.bitcast_convert_type(a, jnp.uint32)
    return (u + 0x7FFF + ((u >> 16) & 1)) >> 16


def _t2_body(p0_ref, p1_ref, hp_ref, dinv_ref, bg_ref, wgat_ref, as_ref,
             ad_ref, gcn_ref, xh_ref, xq_ref, alcs_ref, alcd_ref):
    gsum = p0_ref[...] + p1_ref[...] + hp_ref[...]
    gcn = jnp.maximum(gsum * dinv_ref[...] + bg_ref[...], 0.0)
    gcn_ref[...] = gcn
    xh = jnp.dot(gcn, wgat_ref[...], preferred_element_type=f32)
    xh_ref[...] = xh
    qcols = []
    for k in range(HEADS * D // 32):
        a = _rne_bf16(xh[:, 32 * k:32 * k + 16])
        b = _rne_bf16(xh[:, 32 * k + 16:32 * k + 32])
        qcols.append(lax.bitcast_convert_type(a | (b << 16), f32))
    xq_ref[...] = jnp.concatenate(qcols, axis=1)
    for aref, oref in ((as_ref, alcs_ref), (ad_ref, alcd_ref)):
        a = aref[...]
        cols = []
        for h in range(HEADS):
            seg = xh[:, h * D:(h + 1) * D] * a[:, h * D:(h + 1) * D]
            cols.append(jnp.sum(seg, axis=1, keepdims=True))
        al = jnp.concatenate(cols, axis=1)
        oref[...] = jnp.concatenate([al, jnp.zeros_like(al)], axis=1)


def _t2(p0, p1, hp, dinv2d, bg, wgat, asf, adf):
    return pl.pallas_call(
        _t2_body,
        grid=(NBT,),
        in_specs=[
            pl.BlockSpec((NB, D), lambda i: (i, 0)),
            pl.BlockSpec((NB, D), lambda i: (i, 0)),
            pl.BlockSpec((NB, D), lambda i: (i, 0)),
            pl.BlockSpec((NB, 1), lambda i: (i, 0)),
            pl.BlockSpec((1, D), lambda i: (0, 0)),
            pl.BlockSpec((D, HEADS * D), lambda i: (0, 0)),
            pl.BlockSpec((1, HEADS * D), lambda i: (0, 0)),
            pl.BlockSpec((1, HEADS * D), lambda i: (0, 0)),
        ],
        out_specs=[
            pl.BlockSpec((NB, D), lambda i: (i, 0)),
            pl.BlockSpec((NB, HEADS * D), lambda i: (i, 0)),
            pl.BlockSpec((NB, HEADS * D // 2), lambda i: (i, 0)),
            pl.BlockSpec((NB, 16), lambda i: (i, 0)),
            pl.BlockSpec((NB, 16), lambda i: (i, 0)),
        ],
        out_shape=[
            jax.ShapeDtypeStruct((N, D), f32),
            jax.ShapeDtypeStruct((N, HEADS * D), f32),
            jax.ShapeDtypeStruct((N, HEADS * D // 2), f32),
            jax.ShapeDtypeStruct((N, 16), f32),
            jax.ShapeDtypeStruct((N, 16), f32),
        ],
    )(p0, p1, hp, dinv2d, bg, wgat, asf, adf)


def _t3_body(d0_ref, d1_ref, alcs_ref, alcd_ref, xh_ref, rdenc_ref,
             self_ref):
    z = alcs_ref[...][:, :HEADS] + alcd_ref[...][:, :HEADS]
    ex_self = jnp.exp(jnp.maximum(z, 0.2 * z))
    den = d0_ref[...][:, :HEADS] + d1_ref[...][:, :HEADS] + ex_self
    rden = 1.0 / (den + 1e-16)
    rdenc_ref[...] = jnp.concatenate([rden, jnp.zeros_like(rden)], axis=1)
    w = ex_self * rden
    xh = xh_ref[...]
    st = jnp.zeros((xh.shape[0], D), f32)
    for h in range(HEADS):
        st = st + xh[:, h * D:(h + 1) * D] * w[:, h:h + 1]
    self_ref[...] = st


def _t3(d0, d1, alcs, alcd, xh):
    return pl.pallas_call(
        _t3_body,
        grid=(NBT,),
        in_specs=[
            pl.BlockSpec((NB, 16), lambda i: (i, 0)),
            pl.BlockSpec((NB, 16), lambda i: (i, 0)),
            pl.BlockSpec((NB, 16), lambda i: (i, 0)),
            pl.BlockSpec((NB, 16), lambda i: (i, 0)),
            pl.BlockSpec((NB, HEADS * D), lambda i: (i, 0)),
        ],
        out_specs=[
            pl.BlockSpec((NB, 16), lambda i: (i, 0)),
            pl.BlockSpec((NB, D), lambda i: (i, 0)),
        ],
        out_shape=[
            jax.ShapeDtypeStruct((N, 16), f32),
            jax.ShapeDtypeStruct((N, D), f32),
        ],
    )(d0, d1, alcs, alcd, xh)


def _t4_body(g0_ref, g1_ref, st_ref, bgat_ref, s0_ref, s1_ref, rcnt_ref,
             gcn_ref, w1l_ref, w1r_ref, b1_ref, gat_ref, h1_ref):
    gat_ref[...] = (g0_ref[...] + g1_ref[...] + st_ref[...]) * (1.0 / HEADS) \
        + bgat_ref[...]
    agg1 = (s0_ref[...] + s1_ref[...]) * rcnt_ref[...]
    h1 = jnp.dot(agg1, w1l_ref[...], preferred_element_type=f32) \
        + jnp.dot(gcn_ref[...], w1r_ref[...], preferred_element_type=f32) \
        + b1_ref[...]
    h1_ref[...] = jnp.maximum(h1, 0.0)


def _t4(g0, g1, st, bgat, s0, s1, rcnt2d, gcn, w1l, w1r, b1):
    return pl.pallas_call(
        _t4_body,
        grid=(NBT,),
        in_specs=[
            pl.BlockSpec((NB, D), lambda i: (i, 0)),
            pl.BlockSpec((NB, D), lambda i: (i, 0)),
            pl.BlockSpec((NB, D), lambda i: (i, 0)),
            pl.BlockSpec((1, D), lambda i: (0, 0)),
            pl.BlockSpec((NB, D), lambda i: (i, 0)),
            pl.BlockSpec((NB, D), lambda i: (i, 0)),
            pl.BlockSpec((NB, 1), lambda i: (i, 0)),
            pl.BlockSpec((NB, D), lambda i: (i, 0)),
            pl.BlockSpec((D, D), lambda i: (0, 0)),
            pl.BlockSpec((D, D), lambda i: (0, 0)),
            pl.BlockSpec((1, D), lambda i: (0, 0)),
        ],
        out_specs=[
            pl.BlockSpec((NB, D), lambda i: (i, 0)),
            pl.BlockSpec((NB, D), lambda i: (i, 0)),
        ],
        out_shape=[
            jax.ShapeDtypeStruct((N, D), f32),
            jax.ShapeDtypeStruct((N, D), f32),
        ],
    )(g0, g1, st, bgat, s0, s1, rcnt2d, gcn, w1l, w1r, b1)


def _t5_body(s0_ref, s1_ref, rcnt_ref, h1_ref, w2l_ref, w2r_ref, b2_ref,
             gat_ref, wft_ref, wfb_ref, bf_ref, batch_ref, out_ref,
             ssum, scnt):
    pid = pl.program_id(0)
    agg2 = (s0_ref[...] + s1_ref[...]) * rcnt_ref[...]
    sage = jnp.dot(agg2, w2l_ref[...], preferred_element_type=f32) \
        + jnp.dot(h1_ref[...], w2r_ref[...], preferred_element_type=f32) \
        + b2_ref[...]
    ge = jnp.dot(gat_ref[...], wft_ref[...], preferred_element_type=f32) \
        + jnp.dot(sage, wfb_ref[...], preferred_element_type=f32) \
        + bf_ref[...]
    iota_row = lax.broadcasted_iota(i32, (1, NUM_GRAPHS), 1)
    onehot = (batch_ref[...] == iota_row).astype(f32)
    dn = (((0,), (0,)), ((), ()))
    ps = lax.dot_general(onehot, ge, dn, preferred_element_type=f32)
    pc = lax.dot_general(onehot, jnp.ones_like(ge), dn,
                         preferred_element_type=f32)

    @pl.when(pid == 0)
    def _():
        ssum[...] = jnp.zeros_like(ssum)
        scnt[...] = jnp.zeros_like(scnt)

    ssum[...] += ps
    scnt[...] += pc

    @pl.when(pid == NBT - 1)
    def _():
        out_ref[...] = ssum[...] / jnp.maximum(scnt[...], 1.0)


def _t5(s0, s1, rcnt2d, h1, w2l, w2r, b2, gat, wft, wfb, bf, batch2d):
    return pl.pallas_call(
        _t5_body,
        grid=(NBT,),
        in_specs=[
            pl.BlockSpec((NB, D), lambda i: (i, 0)),
            pl.BlockSpec((NB, D), lambda i: (i, 0)),
            pl.BlockSpec((NB, 1), lambda i: (i, 0)),
            pl.BlockSpec((NB, D), lambda i: (i, 0)),
            pl.BlockSpec((D, D), lambda i: (0, 0)),
            pl.BlockSpec((D, D), lambda i: (0, 0)),
            pl.BlockSpec((1, D), lambda i: (0, 0)),
            pl.BlockSpec((NB, D), lambda i: (i, 0)),
            pl.BlockSpec((D, D), lambda i: (0, 0)),
            pl.BlockSpec((D, D), lambda i: (0, 0)),
            pl.BlockSpec((1, D), lambda i: (0, 0)),
            pl.BlockSpec((NB, 1), lambda i: (i, 0)),
        ],
        out_specs=pl.BlockSpec((NUM_GRAPHS, D), lambda i: (0, 0)),
        out_shape=jax.ShapeDtypeStruct((NUM_GRAPHS, D), f32),
        scratch_shapes=[
            pltpu.VMEM((NUM_GRAPHS, D), f32),
            pltpu.VMEM((NUM_GRAPHS, D), f32),
        ],
    )(s0, s1, rcnt2d, h1, w2l, w2r, b2, gat, wft, wfb, bf, batch2d)


def kernel(x, edge_index, batch, W_gcn, b_gcn, W_gat, a_src, a_dst, b_gat,
           W1_l, W1_r, b1, W2_l, W2_r, b2, W_fin, b_fin):
    src = edge_index[0]
    dst = edge_index[1]
    src3 = src.reshape(NW, NBLK, B)
    dst3 = dst.reshape(NW, NBLK, B)
    srcG = src.reshape(NW, NBLKG, BG)
    dstG = dst.reshape(NW, NBLKG, BG)
    zerosND = jnp.zeros((N, D), f32)
    zerosN16 = jnp.zeros((N, 16), f32)

    degp = _sc_deg(dst3, zerosN16)
    hp, dinv2d, rcnt2d = _t1(degp[0], degp[1], x, W_gcn)
    gcnp = _sc_rows(hp, src3, dst3, zerosND)
    gcn_out, xh, xq, alcs, alcd = _t2(gcnp[0], gcnp[1], hp, dinv2d,
                                  b_gcn.reshape(1, D), W_gat,
                                  a_src.reshape(1, HEADS * D),
                                  a_dst.reshape(1, HEADS * D))

    denp, ex = _sc_gata(alcs, alcd, src3, dst3, zerosN16)
    rdenc, selfterm = _t3(denp[0], denp[1], alcs, alcd, xh)
    wrows = _sc_gatb(xq, ex, rdenc, srcG, dstG)
    iota3 = jnp.arange(E, dtype=i32).reshape(NW, NBLK, B)
    gatp = _sc_rows(wrows, iota3, dst3, zerosND)

    sage1p = _sc_rows(gcn_out, src3, dst3, zerosND)
    gat_out, h1 = _t4(gatp[0], gatp[1], selfterm, b_gat.reshape(1, D),
                      sage1p[0], sage1p[1], rcnt2d, gcn_out, W1_l, W1_r,
                      b1.reshape(1, D))

    sage2p = _sc_rows(h1, src3, dst3, zerosND)
    return _t5(sage2p[0], sage2p[1], rcnt2d, h1, W2_l, W2_r,
               b2.reshape(1, D), gat_out, W_fin[:D], W_fin[D:],
               b_fin.reshape(1, D), batch.reshape(N, 1))


# speedup vs baseline: 1.0481x; 1.0481x over previous
import functools
import jax
import jax.numpy as jnp
from jax import lax
from jax.experimental import pallas as pl
from jax.experimental.pallas import tpu as pltpu
from jax.experimental.pallas import tpu_sc as plsc

N = 10000
E = 320000
D = 128
HEADS = 8
NUM_GRAPHS = 16

NC = 2
NS = 16
NW = NC * NS
EPW = E // NW
B = 100
NBLK = EPW // B
BG = 50
NBLKG = EPW // BG

f32 = jnp.float32
i32 = jnp.int32

_mesh = plsc.VectorSubcoreMesh(core_axis_name="c", subcore_axis_name="s")
_sc_params = pltpu.CompilerParams(use_tc_tiling_on_sc=False)


def _wid():
    return lax.axis_index("s") * NC + lax.axis_index("c")


def _gwait(src, dst, sem):
    pltpu.make_async_copy(src, dst, sem).wait()


@functools.partial(
    pl.kernel,
    out_type=jax.ShapeDtypeStruct((NC, N, 16), f32),
    mesh=_mesh,
    compiler_params=_sc_params,
    scratch_types=[
        pltpu.VMEM_SHARED((N, 16), f32),
        pltpu.VMEM((NBLK, B), i32),
        pltpu.VMEM((B, 16), f32),
        pltpu.SemaphoreType.DMA,
    ],
)
def _sc_deg(dst_hbm, z_hbm, out_hbm, acc, didx, onesb, sem):
    c = lax.axis_index("c")
    s = lax.axis_index("s")
    wid = _wid()

    @pl.when(s == 0)
    def _():
        pltpu.sync_copy(z_hbm, acc)

    pltpu.sync_copy(dst_hbm.at[wid], didx)
    row = jnp.where(lax.iota(i32, 16) == 0, 1.0, 0.0).astype(f32)

    def bstep(j, _):
        onesb[j] = row
        return 0

    lax.fori_loop(0, B, bstep, 0)
    plsc.subcore_barrier()

    def step(i, _):
        for t in range(10):
            pltpu.async_copy(onesb, acc.at[didx.at[i * 10 + t]], sem,
                             add=True)
        for t in range(10):
            _gwait(onesb, acc.at[didx.at[i * 10 + t]], sem)
        return 0

    lax.fori_loop(0, NBLK // 10, step, 0)
    plsc.subcore_barrier()

    @pl.when(s == 0)
    def _():
        pltpu.sync_copy(acc, out_hbm.at[c])


@functools.partial(
    pl.kernel,
    out_type=jax.ShapeDtypeStruct((NC, N, D), f32),
    mesh=_mesh,
    compiler_params=_sc_params,
    scratch_types=[
        pltpu.VMEM_SHARED((N, D), f32),
        pltpu.VMEM((NBLK, B), i32),
        pltpu.VMEM((NBLK, B), i32),
        pltpu.VMEM((B, D), f32),
        pltpu.VMEM((B, D), f32),
        pltpu.SemaphoreType.DMA,
        pltpu.SemaphoreType.DMA,
    ],
)
def _sc_rows(tbl_hbm, src_hbm, dst_hbm, z_hbm, out_hbm, acc, sidx, didx,
             rowsA, rowsB, semA, semB):
    c = lax.axis_index("c")
    s = lax.axis_index("s")
    wid = _wid()

    @pl.when(s == 0)
    def _():
        pltpu.sync_copy(z_hbm, acc)

    pltpu.sync_copy(src_hbm.at[wid], sidx)
    pltpu.sync_copy(dst_hbm.at[wid], didx)
    plsc.subcore_barrier()

    pltpu.async_copy(tbl_hbm.at[sidx.at[0]], rowsA, semA)

    def step(p, _):
        b0 = 2 * p
        b1 = b0 + 1
        pltpu.async_copy(tbl_hbm.at[sidx.at[b1]], rowsB, semB)
        _gwait(tbl_hbm.at[sidx.at[b0]], rowsA, semA)
        pltpu.sync_copy(rowsA, acc.at[didx.at[b0]], add=True)

        @pl.when(b0 + 2 < NBLK)
        def _():
            pltpu.async_copy(tbl_hbm.at[sidx.at[b0 + 2]], rowsA, semA)

        _gwait(tbl_hbm.at[sidx.at[b1]], rowsB, semB)
        pltpu.sync_copy(rowsB, acc.at[didx.at[b1]], add=True)
        return 0

    lax.fori_loop(0, NBLK // 2, step, 0)
    plsc.subcore_barrier()

    @pl.when(s == 0)
    def _():
        pltpu.sync_copy(acc, out_hbm.at[c])


@functools.partial(
    pl.kernel,
    out_type=(
        jax.ShapeDtypeStruct((NC, N, 16), f32),
        jax.ShapeDtypeStruct((E, 16), f32),
    ),
    mesh=_mesh,
    compiler_params=_sc_params,
    scratch_types=[
        pltpu.VMEM_SHARED((N, 16), f32),
        pltpu.VMEM((NBLK, B), i32),
        pltpu.VMEM((NBLK, B), i32),
        pltpu.VMEM((B, 16), f32),
        pltpu.VMEM((B, 16), f32),
        pltpu.VMEM((B, 16), f32),
        pltpu.VMEM((B, 16), f32),
        pltpu.VMEM((B, 16), f32),
        pltpu.SemaphoreType.DMA,
        pltpu.SemaphoreType.DMA,
        pltpu.SemaphoreType.DMA,
        pltpu.SemaphoreType.DMA,
    ],
)
def _sc_gata(alcs_hbm, alcd_hbm, src_hbm, dst_hbm, z_hbm, den_hbm, ex_hbm,
             den, sidx, didx, rsA, rdA, rsB, rdB, exb, semSA, semDA, semSB,
             semDB):
    c = lax.axis_index("c")
    s = lax.axis_index("s")
    wid = _wid()

    @pl.when(s == 0)
    def _():
        pltpu.sync_copy(z_hbm, den)

    pltpu.sync_copy(src_hbm.at[wid], sidx)
    pltpu.sync_copy(dst_hbm.at[wid], didx)
    plsc.subcore_barrier()

    lo_mask = lax.iota(i32, 16) < 8

    def start(b, rs, rd, semS, semD):
        pltpu.async_copy(alcs_hbm.at[sidx.at[b]], rs, semS)
        pltpu.async_copy(alcd_hbm.at[didx.at[b]], rd, semD)

    def finish(b, rs, rd, semS, semD):
        _gwait(alcs_hbm.at[sidx.at[b]], rs, semS)
        _gwait(alcd_hbm.at[didx.at[b]], rd, semD)

        def estep(j, _):
            z = rs[j] + rd[j]
            z = jnp.maximum(z, 0.2 * z)
            exv = jnp.exp(z)
            exb[j] = jnp.where(lo_mask, exv, 0.0)
            return 0

        lax.fori_loop(0, B, estep, 0)
        pltpu.sync_copy(exb, ex_hbm.at[pl.ds((wid * NBLK + b) * B, B)])
        pltpu.sync_copy(exb, den.at[didx.at[b]], add=True)

    start(0, rsA, rdA, semSA, semDA)

    def step(p, _):
        b0 = 2 * p
        b1 = b0 + 1
        start(b1, rsB, rdB, semSB, semDB)
        finish(b0, rsA, rdA, semSA, semDA)

        @pl.when(b0 + 2 < NBLK)
        def _():
            start(b0 + 2, rsA, rdA, semSA, semDA)

        finish(b1, rsB, rdB, semSB, semDB)
        return 0

    lax.fori_loop(0, NBLK // 2, step, 0)
    plsc.subcore_barrier()

    @pl.when(s == 0)
    def _():
        pltpu.sync_copy(den, den_hbm.at[c])


@functools.partial(
    pl.kernel,
    out_type=jax.ShapeDtypeStruct((E, D), f32),
    mesh=_mesh,
    compiler_params=_sc_params,
    scratch_types=[
        pltpu.VMEM((NBLKG, BG), i32),
        pltpu.VMEM((NBLKG, BG), i32),
        pltpu.VMEM((BG, HEADS * D // 2), f32),
        pltpu.VMEM((BG, HEADS * D // 2), f32),
        pltpu.VMEM((BG, 16), f32),
        pltpu.VMEM((BG, 16), f32),
        pltpu.VMEM((BG, 16), f32),
        pltpu.VMEM((BG, 16), f32),
        pltpu.VMEM((BG, D), f32),
        pltpu.VMEM((BG, D), f32),
        pltpu.SemaphoreType.DMA,
        pltpu.SemaphoreType.DMA,
        pltpu.SemaphoreType.DMA,
        pltpu.SemaphoreType.DMA,
        pltpu.SemaphoreType.DMA,
        pltpu.SemaphoreType.DMA,
        pltpu.SemaphoreType.DMA,
        pltpu.SemaphoreType.DMA,
    ],
)
def _sc_gatb(xh_hbm, ex_hbm, rdenc_hbm, src_hbm, dst_hbm, wr_hbm,
             sidx, didx, xrA, xrB, exA, exB, rdA, rdB, outbA, outbB,
             semXA, semXB, semEA, semEB, semRA, semRB, semWA, semWB):
    wid = _wid()

    pltpu.sync_copy(src_hbm.at[wid], sidx)
    pltpu.sync_copy(dst_hbm.at[wid], didx)

    def start(b, xr, exb, rdb, semX, semE, semR):
        pltpu.async_copy(xh_hbm.at[sidx.at[b]], xr, semX)
        pltpu.async_copy(ex_hbm.at[pl.ds((wid * NBLKG + b) * BG, BG)], exb,
                         semE)
        pltpu.async_copy(rdenc_hbm.at[didx.at[b]], rdb, semR)

    hidx = [jnp.full((16, 1), h, i32) for h in range(HEADS)]
    _gdn = lax.GatherDimensionNumbers(offset_dims=(),
                                      collapsed_slice_dims=(0,),
                                      start_index_map=(0,))

    def _splat(v, idx):
        return lax.gather(v, idx, _gdn, (1,),
                          mode=lax.GatherScatterMode.PROMISE_IN_BOUNDS)

    def finish(b, xr, exb, rdb, outb, semX, semE, semR, semW):
        _gwait(xh_hbm.at[sidx.at[b]], xr, semX)
        _gwait(ex_hbm.at[pl.ds((wid * NBLKG + b) * BG, BG)], exb, semE)
        _gwait(rdenc_hbm.at[didx.at[b]], rdb, semR)

        @pl.when(b >= 2)
        def _():
            _gwait(outb, wr_hbm.at[pl.ds((wid * NBLKG + b) * BG, BG)], semW)

        def estep(j, _):
            cvec = exb[j] * rdb[j]
            accs = [jnp.zeros((16,), f32) for _ in range(D // 16)]
            for h in range(HEADS):
                cv = _splat(cvec, hidx[h])
                for k2 in range(D // 32):
                    w = xr[j, pl.ds(h * (D // 2) + k2 * 16, 16)]
                    wi = lax.bitcast_convert_type(w, i32)
                    va = lax.bitcast_convert_type(wi << 16, f32)
                    vb = lax.bitcast_convert_type(wi & jnp.int32(-65536), f32)
                    accs[2 * k2] = accs[2 * k2] + cv * va
                    accs[2 * k2 + 1] = accs[2 * k2 + 1] + cv * vb
            for k in range(D // 16):
                outb[j, pl.ds(k * 16, 16)] = accs[k]
            return 0

        lax.fori_loop(0, BG, estep, 0)
        pltpu.async_copy(outb, wr_hbm.at[pl.ds((wid * NBLKG + b) * BG, BG)],
                         semW)

    start(0, xrA, exA, rdA, semXA, semEA, semRA)

    def step(p, _):
        b0 = 2 * p
        b1 = b0 + 1
        start(b1, xrB, exB, rdB, semXB, semEB, semRB)
        finish(b0, xrA, exA, rdA, outbA, semXA, semEA, semRA, semWA)

        @pl.when(b0 + 2 < NBLKG)
        def _():
            start(b0 + 2, xrA, exA, rdA, semXA, semEA, semRA)

        finish(b1, xrB, exB, rdB, outbB, semXB, semEB, semRB, semWB)
        return 0

    lax.fori_loop(0, NBLKG // 2, step, 0)
    _gwait(outbA, wr_hbm.at[pl.ds(wid * EPW, BG)], semWA)
    _gwait(outbB, wr_hbm.at[pl.ds(wid * EPW, BG)], semWB)


NB = 2000
NBT = N // NB


def _t1_body(d0_ref, d1_ref, x_ref, w_ref, hp_ref, dinv_ref, rcnt_ref):
    cnt = d0_ref[...][:, :1] + d1_ref[...][:, :1]
    dinv = lax.rsqrt(cnt + 1.0)
    rcnt_ref[...] = 1.0 / jnp.maximum(cnt, 1.0)
    h = jnp.dot(x_ref[...], w_ref[...], preferred_element_type=f32)
    hp_ref[...] = h * dinv
    dinv_ref[...] = dinv


def _t1(d0, d1, x, w_gcn):
    return pl.pallas_call(
        _t1_body,
        grid=(NBT,),
        in_specs=[
            pl.BlockSpec((NB, 16), lambda i: (i, 0)),
            pl.BlockSpec((NB, 16), lambda i: (i, 0)),
            pl.BlockSpec((NB, D), lambda i: (i, 0)),
            pl.BlockSpec((D, D), lambda i: (0, 0)),
        ],
        out_specs=[
            pl.BlockSpec((NB, D), lambda i: (i, 0)),
            pl.BlockSpec((NB, 1), lambda i: (i, 0)),
            pl.BlockSpec((NB, 1), lambda i: (i, 0)),
        ],
        out_shape=[
            jax.ShapeDtypeStruct((N, D), f32),
            jax.ShapeDtypeStruct((N, 1), f32),
            jax.ShapeDtypeStruct((N, 1), f32),
        ],
    )(d0, d1, x, w_gcn)


def _rne_bf16(a):
    u = lax.bitcast_convert_type(a, jnp.uint32)
    return (u + 0x7FFF + ((u >> 16) & 1)) >> 16


def _t2_body(p0_ref, p1_ref, hp_ref, dinv_ref, bg_ref, wgat_ref, as_ref,
             ad_ref, gcn_ref, xh_ref, xq_ref, alcs_ref, alcd_ref):
    gsum = p0_ref[...] + p1_ref[...] + hp_ref[...]
    gcn = jnp.maximum(gsum * dinv_ref[...] + bg_ref[...], 0.0)
    gcn_ref[...] = gcn
    xh = jnp.dot(gcn, wgat_ref[...], preferred_element_type=f32)
    xh_ref[...] = xh
    qcols = []
    for k in range(HEADS * D // 32):
        a = _rne_bf16(xh[:, 32 * k:32 * k + 16])
        b = _rne_bf16(xh[:, 32 * k + 16:32 * k + 32])
        qcols.append(lax.bitcast_convert_type(a | (b << 16), f32))
    xq_ref[...] = jnp.concatenate(qcols, axis=1)
    for aref, oref in ((as_ref, alcs_ref), (ad_ref, alcd_ref)):
        a = aref[...]
        cols = []
        for h in range(HEADS):
            seg = xh[:, h * D:(h + 1) * D] * a[:, h * D:(h + 1) * D]
            cols.append(jnp.sum(seg, axis=1, keepdims=True))
        al = jnp.concatenate(cols, axis=1)
        oref[...] = jnp.concatenate([al, jnp.zeros_like(al)], axis=1)


def _t2(p0, p1, hp, dinv2d, bg, wgat, asf, adf):
    return pl.pallas_call(
        _t2_body,
        grid=(NBT,),
        in_specs=[
            pl.BlockSpec((NB, D), lambda i: (i, 0)),
            pl.BlockSpec((NB, D), lambda i: (i, 0)),
            pl.BlockSpec((NB, D), lambda i: (i, 0)),
            pl.BlockSpec((NB, 1), lambda i: (i, 0)),
            pl.BlockSpec((1, D), lambda i: (0, 0)),
            pl.BlockSpec((D, HEADS * D), lambda i: (0, 0)),
            pl.BlockSpec((1, HEADS * D), lambda i: (0, 0)),
            pl.BlockSpec((1, HEADS * D), lambda i: (0, 0)),
        ],
        out_specs=[
            pl.BlockSpec((NB, D), lambda i: (i, 0)),
            pl.BlockSpec((NB, HEADS * D), lambda i: (i, 0)),
            pl.BlockSpec((NB, HEADS * D // 2), lambda i: (i, 0)),
            pl.BlockSpec((NB, 16), lambda i: (i, 0)),
            pl.BlockSpec((NB, 16), lambda i: (i, 0)),
        ],
        out_shape=[
            jax.ShapeDtypeStruct((N, D), f32),
            jax.ShapeDtypeStruct((N, HEADS * D), f32),
            jax.ShapeDtypeStruct((N, HEADS * D // 2), f32),
            jax.ShapeDtypeStruct((N, 16), f32),
            jax.ShapeDtypeStruct((N, 16), f32),
        ],
    )(p0, p1, hp, dinv2d, bg, wgat, asf, adf)


def _t3_body(d0_ref, d1_ref, alcs_ref, alcd_ref, xh_ref, rdenc_ref,
             self_ref):
    z = alcs_ref[...][:, :HEADS] + alcd_ref[...][:, :HEADS]
    ex_self = jnp.exp(jnp.maximum(z, 0.2 * z))
    den = d0_ref[...][:, :HEADS] + d1_ref[...][:, :HEADS] + ex_self
    rden = 1.0 / (den + 1e-16)
    rdenc_ref[...] = jnp.concatenate([rden, jnp.zeros_like(rden)], axis=1)
    w = ex_self * rden
    xh = xh_ref[...]
    st = jnp.zeros((xh.shape[0], D), f32)
    for h in range(HEADS):
        st = st + xh[:, h * D:(h + 1) * D] * w[:, h:h + 1]
    self_ref[...] = st


def _t3(d0, d1, alcs, alcd, xh):
    return pl.pallas_call(
        _t3_body,
        grid=(NBT,),
        in_specs=[
            pl.BlockSpec((NB, 16), lambda i: (i, 0)),
            pl.BlockSpec((NB, 16), lambda i: (i, 0)),
            pl.BlockSpec((NB, 16), lambda i: (i, 0)),
            pl.BlockSpec((NB, 16), lambda i: (i, 0)),
            pl.BlockSpec((NB, HEADS * D), lambda i: (i, 0)),
        ],
        out_specs=[
            pl.BlockSpec((NB, 16), lambda i: (i, 0)),
            pl.BlockSpec((NB, D), lambda i: (i, 0)),
        ],
        out_shape=[
            jax.ShapeDtypeStruct((N, 16), f32),
            jax.ShapeDtypeStruct((N, D), f32),
        ],
    )(d0, d1, alcs, alcd, xh)


def _t4_body(g0_ref, g1_ref, st_ref, bgat_ref, s0_ref, s1_ref, rcnt_ref,
             gcn_ref, w1l_ref, w1r_ref, b1_ref, gat_ref, h1_ref):
    gat_ref[...] = (g0_ref[...] + g1_ref[...] + st_ref[...]) * (1.0 / HEADS) \
        + bgat_ref[...]
    agg1 = (s0_ref[...] + s1_ref[...]) * rcnt_ref[...]
    h1 = jnp.dot(agg1, w1l_ref[...], preferred_element_type=f32) \
        + jnp.dot(gcn_ref[...], w1r_ref[...], preferred_element_type=f32) \
        + b1_ref[...]
    h1_ref[...] = jnp.maximum(h1, 0.0)


def _t4(g0, g1, st, bgat, s0, s1, rcnt2d, gcn, w1l, w1r, b1):
    return pl.pallas_call(
        _t4_body,
        grid=(NBT,),
        in_specs=[
            pl.BlockSpec((NB, D), lambda i: (i, 0)),
            pl.BlockSpec((NB, D), lambda i: (i, 0)),
            pl.BlockSpec((NB, D), lambda i: (i, 0)),
            pl.BlockSpec((1, D), lambda i: (0, 0)),
            pl.BlockSpec((NB, D), lambda i: (i, 0)),
            pl.BlockSpec((NB, D), lambda i: (i, 0)),
            pl.BlockSpec((NB, 1), lambda i: (i, 0)),
            pl.BlockSpec((NB, D), lambda i: (i, 0)),
            pl.BlockSpec((D, D), lambda i: (0, 0)),
            pl.BlockSpec((D, D), lambda i: (0, 0)),
            pl.BlockSpec((1, D), lambda i: (0, 0)),
        ],
        out_specs=[
            pl.BlockSpec((NB, D), lambda i: (i, 0)),
            pl.BlockSpec((NB, D), lambda i: (i, 0)),
        ],
        out_shape=[
            jax.ShapeDtypeStruct((N, D), f32),
            jax.ShapeDtypeStruct((N, D), f32),
        ],
    )(g0, g1, st, bgat, s0, s1, rcnt2d, gcn, w1l, w1r, b1)


def _t5_body(s0_ref, s1_ref, rcnt_ref, h1_ref, w2l_ref, w2r_ref, b2_ref,
             gat_ref, wft_ref, wfb_ref, bf_ref, batch_ref, out_ref,
             ssum, scnt):
    pid = pl.program_id(0)
    agg2 = (s0_ref[...] + s1_ref[...]) * rcnt_ref[...]
    sage = jnp.dot(agg2, w2l_ref[...], preferred_element_type=f32) \
        + jnp.dot(h1_ref[...], w2r_ref[...], preferred_element_type=f32) \
        + b2_ref[...]
    ge = jnp.dot(gat_ref[...], wft_ref[...], preferred_element_type=f32) \
        + jnp.dot(sage, wfb_ref[...], preferred_element_type=f32) \
        + bf_ref[...]
    iota_row = lax.broadcasted_iota(i32, (1, NUM_GRAPHS), 1)
    onehot = (batch_ref[...] == iota_row).astype(f32)
    dn = (((0,), (0,)), ((), ()))
    ps = lax.dot_general(onehot, ge, dn, preferred_element_type=f32)
    pc = lax.dot_general(onehot, jnp.ones_like(ge), dn,
                         preferred_element_type=f32)

    @pl.when(pid == 0)
    def _():
        ssum[...] = jnp.zeros_like(ssum)
        scnt[...] = jnp.zeros_like(scnt)

    ssum[...] += ps
    scnt[...] += pc

    @pl.when(pid == NBT - 1)
    def _():
        out_ref[...] = ssum[...] / jnp.maximum(scnt[...], 1.0)


def _t5(s0, s1, rcnt2d, h1, w2l, w2r, b2, gat, wft, wfb, bf, batch2d):
    return pl.pallas_call(
        _t5_body,
        grid=(NBT,),
        in_specs=[
            pl.BlockSpec((NB, D), lambda i: (i, 0)),
            pl.BlockSpec((NB, D), lambda i: (i, 0)),
            pl.BlockSpec((NB, 1), lambda i: (i, 0)),
            pl.BlockSpec((NB, D), lambda i: (i, 0)),
            pl.BlockSpec((D, D), lambda i: (0, 0)),
            pl.BlockSpec((D, D), lambda i: (0, 0)),
            pl.BlockSpec((1, D), lambda i: (0, 0)),
            pl.BlockSpec((NB, D), lambda i: (i, 0)),
            pl.BlockSpec((D, D), lambda i: (0, 0)),
            pl.BlockSpec((D, D), lambda i: (0, 0)),
            pl.BlockSpec((1, D), lambda i: (0, 0)),
            pl.BlockSpec((NB, 1), lambda i: (i, 0)),
        ],
        out_specs=pl.BlockSpec((NUM_GRAPHS, D), lambda i: (0, 0)),
        out_shape=jax.ShapeDtypeStruct((NUM_GRAPHS, D), f32),
        scratch_shapes=[
            pltpu.VMEM((NUM_GRAPHS, D), f32),
            pltpu.VMEM((NUM_GRAPHS, D), f32),
        ],
    )(s0, s1, rcnt2d, h1, w2l, w2r, b2, gat, wft, wfb, bf, batch2d)


def kernel(x, edge_index, batch, W_gcn, b_gcn, W_gat, a_src, a_dst, b_gat,
           W1_l, W1_r, b1, W2_l, W2_r, b2, W_fin, b_fin):
    src = edge_index[0]
    dst = edge_index[1]
    src3 = src.reshape(NW, NBLK, B)
    dst3 = dst.reshape(NW, NBLK, B)
    srcG = src.reshape(NW, NBLKG, BG)
    dstG = dst.reshape(NW, NBLKG, BG)
    zerosND = jnp.zeros((N, D), f32)
    zerosN16 = jnp.zeros((N, 16), f32)

    degp = _sc_deg(dst3, zerosN16)
    hp, dinv2d, rcnt2d = _t1(degp[0], degp[1], x, W_gcn)
    gcnp = _sc_rows(hp, src3, dst3, zerosND)
    gcn_out, xh, xq, alcs, alcd = _t2(gcnp[0], gcnp[1], hp, dinv2d,
                                  b_gcn.reshape(1, D), W_gat,
                                  a_src.reshape(1, HEADS * D),
                                  a_dst.reshape(1, HEADS * D))

    denp, ex = _sc_gata(alcs, alcd, src3, dst3, zerosN16)
    rdenc, selfterm = _t3(denp[0], denp[1], alcs, alcd, xh)
    wrows = _sc_gatb(xq, ex, rdenc, srcG, dstG)
    iota3 = jnp.arange(E, dtype=i32).reshape(NW, NBLK, B)
    gatp = _sc_rows(wrows, iota3, dst3, zerosND)

    sage1p = _sc_rows(gcn_out, src3, dst3, zerosND)
    gat_out, h1 = _t4(gatp[0], gatp[1], selfterm, b_gat.reshape(1, D),
                      sage1p[0], sage1p[1], rcnt2d, gcn_out, W1_l, W1_r,
                      b1.reshape(1, D))

    sage2p = _sc_rows(h1, src3, dst3, zerosND)
    return _t5(sage2p[0], sage2p[1], rcnt2d, h1, W2_l, W2_r,
               b2.reshape(1, D), gat_out, W_fin[:D], W_fin[D:],
               b_fin.reshape(1, D), batch.reshape(N, 1))
